# baseline (speedup 1.0000x reference)
import functools

import jax
import jax.numpy as jnp
from jax import lax
from jax.experimental import pallas as pl
from jax.experimental.pallas import tpu as pltpu
from jax.experimental.pallas import tpu_sc as plsc

NUM_CORES = 2
NUM_SUBCORES = 16
NW = NUM_CORES * NUM_SUBCORES
CH = 64


def _sc_gather(tok_table, ids_2d, n_chunks, hidden):
    mesh = plsc.VectorSubcoreMesh(
        core_axis_name="c", subcore_axis_name="s",
        num_cores=NUM_CORES, num_subcores=NUM_SUBCORES)

    @functools.partial(
        pl.kernel,
        out_type=jax.ShapeDtypeStruct((NW, n_chunks, CH, hidden), jnp.float32),
        mesh=mesh,
        scratch_types=[
            pltpu.VMEM((n_chunks * CH,), jnp.int32),
            pltpu.VMEM((CH, hidden), jnp.float32),
            pltpu.SemaphoreType.DMA,
        ],
    )
    def gather_kernel(table_hbm, ids_hbm, out_hbm, idx_v, buf0, gsem):
        wid = lax.axis_index("s") * NUM_CORES + lax.axis_index("c")
        pltpu.sync_copy(ids_hbm.at[wid], idx_v)

        def body(c, _):
            rows = pl.ds(c * CH, CH)
            pltpu.async_copy(
                table_hbm.at[idx_v.at[rows]], buf0, gsem).wait()
            pltpu.sync_copy(buf0, out_hbm.at[wid, c])
            return 0

        lax.fori_loop(0, n_chunks, body, 0)

    return gather_kernel(tok_table, ids_2d)


def _tc_finish(embed, onehot, pos_slab, combo8, gb, rows_blk, n_total,
               blk_off, prev_out=None):
    n, hidden = embed.shape

    def body(*refs):
        if prev_out is None:
            emb_ref, oh_ref, pos_ref, combo_ref, gb_ref, out_ref = refs
        else:
            _, emb_ref, oh_ref, pos_ref, combo_ref, gb_ref, out_ref = refs
        x = emb_ref[...] + pos_ref[...]
        x = x + lax.dot_general(
            oh_ref[...], combo_ref[...], (((0,), (0,)), ((), ())),
            preferred_element_type=jnp.float32,
            precision=lax.Precision.HIGHEST)
        mean = jnp.mean(x, axis=-1, keepdims=True)
        xc = x - mean
        var = jnp.mean(xc * xc, axis=-1, keepdims=True)
        y = xc * lax.rsqrt(var + 1e-5)
        gbv = gb_ref[...]
        out_ref[...] = y * gbv[0:1, :] + gbv[1:2, :]

    in_specs = [
        pl.BlockSpec((rows_blk, hidden), lambda i: (i, 0)),
        pl.BlockSpec((8, rows_blk), lambda i: (0, i)),
        pl.BlockSpec((rows_blk, hidden), lambda i: (0, 0)),
        pl.BlockSpec((8, hidden), lambda i: (0, 0)),
        pl.BlockSpec((8, hidden), lambda i: (0, 0)),
    ]
    args = [embed, onehot, pos_slab, combo8, gb]
    aliases = {}
    if prev_out is not None:
        in_specs = [pl.BlockSpec((8, hidden), lambda i: (0, 0))] + in_specs
        args = [prev_out] + args
        aliases = {0: 0}

    return pl.pallas_call(
        body,
        grid=(n // rows_blk,),
        in_specs=in_specs,
        out_specs=pl.BlockSpec((rows_blk, hidden),
                               lambda i: (i + blk_off, 0)),
        out_shape=jax.ShapeDtypeStruct((n_total, hidden), jnp.float32),
        input_output_aliases=aliases,
    )(*args)


def kernel(input_ids, segment_ids, tok_table, pos_table, seg_table, gamma,
           beta):
    bsz, sent_len = input_ids.shape
    hidden = tok_table.shape[1]
    n_tok = bsz * sent_len
    assert n_tok % (NW * CH) == 0
    n_chunks = n_tok // (NW * CH)
    per_w = n_chunks * CH

    n_slices = 2
    n_sl = n_tok // n_slices
    ids_sl = input_ids.reshape(n_slices, NW, per_w // n_slices)
    gathered = [
        _sc_gather(tok_table, ids_sl[s], n_chunks // n_slices, hidden)
        for s in range(n_slices)
    ]

    tok0 = tok_table[0:1]
    combo8 = jnp.concatenate(
        [seg_table[0:1], seg_table[1:2],
         seg_table[0:1] - tok0, seg_table[1:2] - tok0,
         jnp.zeros((4, hidden), jnp.float32)], axis=0)
    combo_idx = (segment_ids + 2 * (input_ids == 0)).reshape(n_tok)
    onehot_t = (jnp.arange(8)[:, None] == combo_idx[None, :]).astype(
        jnp.float32)

    rows_blk = 32 * sent_len
    pos_slab = jnp.tile(pos_table[:sent_len], (32, 1))
    gb = jnp.concatenate(
        [gamma[None, :], beta[None, :], jnp.zeros((6, hidden), jnp.float32)],
        axis=0)
    blks_per_sl = n_sl // rows_blk
    out = None
    for s in range(n_slices):
        out = _tc_finish(
            gathered[s].reshape(n_sl, hidden),
            lax.dynamic_slice_in_dim(onehot_t, s * n_sl, n_sl, 1),
            pos_slab, combo8, gb, rows_blk, n_tok, s * blks_per_sl,
            prev_out=out)
    return out.reshape(bsz, sent_len, hidden)


# speedup vs baseline: 1.2468x; 1.2468x over previous
import functools

import jax
import jax.numpy as jnp
from jax import lax
from jax.experimental import pallas as pl
from jax.experimental.pallas import tpu as pltpu
from jax.experimental.pallas import tpu_sc as plsc

NUM_CORES = 2
NUM_SUBCORES = 16
NW = NUM_CORES * NUM_SUBCORES
CH = 128


def _sc_gather(tok_table, ids_2d, n_chunks, hidden):
    mesh = plsc.VectorSubcoreMesh(
        core_axis_name="c", subcore_axis_name="s",
        num_cores=NUM_CORES, num_subcores=NUM_SUBCORES)

    @functools.partial(
        pl.kernel,
        out_type=jax.ShapeDtypeStruct((NW, n_chunks, CH, hidden), jnp.float32),
        mesh=mesh,
        scratch_types=[
            pltpu.VMEM((n_chunks * CH,), jnp.int32),
            pltpu.VMEM((CH, hidden), jnp.float32),
            pltpu.VMEM((CH, hidden), jnp.float32),
            pltpu.SemaphoreType.DMA,
            pltpu.SemaphoreType.DMA,
            pltpu.SemaphoreType.DMA,
            pltpu.SemaphoreType.DMA,
        ],
    )
    def gather_kernel(table_hbm, ids_hbm, out_hbm, idx_v, buf0, buf1,
                      gsem0, gsem1, osem0, osem1):
        wid = lax.axis_index("s") * NUM_CORES + lax.axis_index("c")
        pltpu.sync_copy(ids_hbm.at[wid], idx_v)
        bufs = (buf0, buf1)
        gsems = (gsem0, gsem1)
        osems = (osem0, osem1)

        def g_rows(c):
            return idx_v.at[pl.ds(c * CH, CH)]

        pltpu.async_copy(table_hbm.at[g_rows(0)], buf0, gsem0)

        def wave(i, _):
            for k in range(2):
                c = i * 2 + k
                buf, gsem, osem = bufs[k], gsems[k], osems[k]
                nbuf, ngsem, nosem = (bufs[1 - k], gsems[1 - k],
                                      osems[1 - k])

                @pl.when(c + 1 < n_chunks)
                def _():
                    @pl.when(c >= 1)
                    def _():
                        pltpu.make_async_copy(
                            nbuf, out_hbm.at[wid, c - 1], nosem).wait()
                    pltpu.async_copy(
                        table_hbm.at[g_rows(c + 1)], nbuf, ngsem)

                pltpu.make_async_copy(
                    table_hbm.at[g_rows(c)], buf, gsem).wait()
                pltpu.async_copy(buf, out_hbm.at[wid, c], osem)
            return 0

        lax.fori_loop(0, n_chunks // 2, wave, 0)
        if n_chunks % 2:
            last = n_chunks - 1
            pltpu.make_async_copy(
                table_hbm.at[g_rows(last)], buf0, gsem0).wait()
            pltpu.async_copy(buf0, out_hbm.at[wid, last], osem0)
            pltpu.make_async_copy(
                buf1, out_hbm.at[wid, last - 1], osem1).wait()
            pltpu.make_async_copy(
                buf0, out_hbm.at[wid, last], osem0).wait()
        else:
            pltpu.make_async_copy(
                buf0, out_hbm.at[wid, n_chunks - 2], osem0).wait()
            pltpu.make_async_copy(
                buf1, out_hbm.at[wid, n_chunks - 1], osem1).wait()

    return gather_kernel(tok_table, ids_2d)


def _tc_finish(embed, onehot, pos_slab, combo8, gb, rows_blk, n_total,
               blk_off, prev_out=None):
    n, hidden = embed.shape

    def body(*refs):
        if prev_out is None:
            emb_ref, oh_ref, pos_ref, combo_ref, gb_ref, out_ref = refs
        else:
            _, emb_ref, oh_ref, pos_ref, combo_ref, gb_ref, out_ref = refs
        x = emb_ref[...] + pos_ref[...]
        x = x + lax.dot_general(
            oh_ref[...], combo_ref[...], (((0,), (0,)), ((), ())),
            preferred_element_type=jnp.float32,
            precision=lax.Precision.HIGHEST)
        mean = jnp.mean(x, axis=-1, keepdims=True)
        xc = x - mean
        var = jnp.mean(xc * xc, axis=-1, keepdims=True)
        y = xc * lax.rsqrt(var + 1e-5)
        gbv = gb_ref[...]
        out_ref[...] = y * gbv[0:1, :] + gbv[1:2, :]

    in_specs = [
        pl.BlockSpec((rows_blk, hidden), lambda i: (i, 0)),
        pl.BlockSpec((8, rows_blk), lambda i: (0, i)),
        pl.BlockSpec((rows_blk, hidden), lambda i: (0, 0)),
        pl.BlockSpec((8, hidden), lambda i: (0, 0)),
        pl.BlockSpec((8, hidden), lambda i: (0, 0)),
    ]
    args = [embed, onehot, pos_slab, combo8, gb]
    aliases = {}
    if prev_out is not None:
        in_specs = [pl.BlockSpec((8, hidden), lambda i: (0, 0))] + in_specs
        args = [prev_out] + args
        aliases = {0: 0}

    return pl.pallas_call(
        body,
        grid=(n // rows_blk,),
        in_specs=in_specs,
        out_specs=pl.BlockSpec((rows_blk, hidden),
                               lambda i: (i + blk_off, 0)),
        out_shape=jax.ShapeDtypeStruct((n_total, hidden), jnp.float32),
        input_output_aliases=aliases,
    )(*args)


def kernel(input_ids, segment_ids, tok_table, pos_table, seg_table, gamma,
           beta):
    bsz, sent_len = input_ids.shape
    hidden = tok_table.shape[1]
    n_tok = bsz * sent_len
    assert n_tok % (NW * CH) == 0
    n_chunks = n_tok // (NW * CH)
    per_w = n_chunks * CH

    n_slices = 2
    n_sl = n_tok // n_slices
    ids_sl = input_ids.reshape(n_slices, NW, per_w // n_slices)
    gathered = [
        _sc_gather(tok_table, ids_sl[s], n_chunks // n_slices, hidden)
        for s in range(n_slices)
    ]

    tok0 = tok_table[0:1]
    combo8 = jnp.concatenate(
        [seg_table[0:1], seg_table[1:2],
         seg_table[0:1] - tok0, seg_table[1:2] - tok0,
         jnp.zeros((4, hidden), jnp.float32)], axis=0)
    combo_idx = (segment_ids + 2 * (input_ids == 0)).reshape(n_tok)
    onehot_t = (jnp.arange(8)[:, None] == combo_idx[None, :]).astype(
        jnp.float32)

    rows_blk = 32 * sent_len
    pos_slab = jnp.tile(pos_table[:sent_len], (32, 1))
    gb = jnp.concatenate(
        [gamma[None, :], beta[None, :], jnp.zeros((6, hidden), jnp.float32)],
        axis=0)
    blks_per_sl = n_sl // rows_blk
    out = None
    for s in range(n_slices):
        out = _tc_finish(
            gathered[s].reshape(n_sl, hidden),
            lax.dynamic_slice_in_dim(onehot_t, s * n_sl, n_sl, 1),
            pos_slab, combo8, gb, rows_blk, n_tok, s * blks_per_sl,
            prev_out=out)
    return out.reshape(bsz, sent_len, hidden)
